# baseline (speedup 1.0000x reference)
import functools

import jax
import jax.numpy as jnp
from jax import lax
from jax.experimental import pallas as pl
from jax.experimental.pallas import tpu as pltpu
from jax.experimental.pallas import tpu_sc as plsc

B = 4096
HIST = 200
D = 64
V = 1000000
NC = 2
NS = 16
NW = NC * NS
BPW = B // NW
CHUNK = 40
NCHUNK = HIST // CHUNK
NLANE = 16
NVREG = D // NLANE
HISTP = 256

BLK = 128
FULLBLK = V // BLK
TAIL = V - FULLBLK * BLK
BLK_PER_W = -(-FULLBLK // NW)


def _tr_body(tt_hbm, out_hbm, in_a, in_b, tail_in, out_a, out_b, tail_out,
             sem_ia, sem_ib, sem_oa, sem_ob):
  cid = lax.axis_index("c")
  sid = lax.axis_index("s")
  wid = sid * NC + cid
  base = wid * BLK_PER_W
  nblk = jnp.minimum(BLK_PER_W, jnp.maximum(0, FULLBLK - base))

  ins = (in_a, in_b)
  outs = (out_a, out_b)
  sem_i = (sem_ia, sem_ib)
  sem_o = (sem_oa, sem_ob)
  lanes = lax.broadcasted_iota(jnp.int32, (NLANE,), 0)

  def start_in(t, s):
    @pl.when(t < nblk)
    def _():
      pltpu.async_copy(tt_hbm.at[:, pl.ds((base + t) * BLK, BLK)],
                       ins[s], sem_i[s])

  def wait_in(s):
    pltpu.make_async_copy(tt_hbm.at[:, pl.ds(0, BLK)], ins[s], sem_i[s]).wait()

  def wait_out(s):
    pltpu.make_async_copy(outs[s], out_hbm.at[pl.ds(0, BLK * D)],
                          sem_o[s]).wait()

  iota64 = D * lanes

  def transpose(s):
    def drow(d):
      vbase = iota64 + ((d + lanes) & 15) + (d - (d & 15))
      for g in range(BLK // NLANE):
        vec = ins[s][d, pl.ds(g * NLANE, NLANE)]
        plsc.store_scatter(outs[s], [vbase + g * (NLANE * D)], vec)
    plsc.parallel_loop(0, D, unroll=16)(drow)

  start_in(0, 0)

  def body(p, _):
    for sbuf in range(2):
      t = 2 * p + sbuf

      @pl.when(t < nblk)
      def _():
        start_in(t + 1, 1 - sbuf)
        wait_in(sbuf)

        @pl.when(t >= 2)
        def _():
          wait_out(sbuf)

        transpose(sbuf)
        pltpu.async_copy(outs[sbuf],
                         out_hbm.at[pl.ds((base + t) * BLK * D, BLK * D)],
                         sem_o[sbuf])
    return 0

  lax.fori_loop(0, (BLK_PER_W + 1) // 2, body, 0)
  wait_out(0)
  wait_out(1)

  @pl.when(wid == NW - 1)
  def _():
    pltpu.sync_copy(tt_hbm.at[:, pl.ds(FULLBLK * BLK, TAIL)], tail_in)

    def drow(d, _):
      vbase = iota64 + ((d + lanes) & 15) + (d - (d & 15))
      for g in range(TAIL // NLANE):
        vec = tail_in[d, pl.ds(g * NLANE, NLANE)]
        plsc.store_scatter(tail_out, [vbase + g * (NLANE * D)], vec)
      return 0
    lax.fori_loop(0, D, drow, 0)
    pltpu.sync_copy(tail_out, out_hbm.at[pl.ds(FULLBLK * BLK * D, TAIL * D)])


@functools.partial(jax.jit, static_argnums=())
def _sc_transpose(table_t):
  mesh = plsc.VectorSubcoreMesh(core_axis_name="c", subcore_axis_name="s")
  return pl.kernel(
      _tr_body,
      out_type=jax.ShapeDtypeStruct((V * D,), jnp.float32),
      mesh=mesh,
      compiler_params=pltpu.CompilerParams(use_tc_tiling_on_sc=True,
                                           needs_layout_passes=False),
      scratch_types=(
          [pltpu.VMEM((D, BLK), jnp.float32) for _ in range(2)]
          + [pltpu.VMEM((D, TAIL), jnp.float32)]
          + [pltpu.VMEM((BLK * D,), jnp.float32) for _ in range(2)]
          + [pltpu.VMEM((TAIL * D,), jnp.float32)]
          + [pltpu.SemaphoreType.DMA] * 4
      ),
  )(table_t)


def _sc_body(idx_hbm, table_hbm, out_hbm, idx_v, *rest):
  rows_flat = rest[:2 * NCHUNK]
  out_v = rest[2 * NCHUNK]
  sems = rest[2 * NCHUNK + 1:]
  rows = (rows_flat[:NCHUNK], rows_flat[NCHUNK:])

  cid = lax.axis_index("c")
  sid = lax.axis_index("s")
  wid = sid * NC + cid
  base = wid * BPW

  pltpu.sync_copy(idx_hbm.at[pl.ds(base, BPW)], idx_v)

  def start_row(i, s):
    for c in range(NCHUNK):
      pltpu.async_copy(
          table_hbm.at[idx_v.at[i, pl.ds(c * CHUNK, CHUNK)]],
          rows[s][c], sems[s])

  def wait_row(s):
    for c in range(NCHUNK):
      pltpu.make_async_copy(
          table_hbm.at[pl.ds(0, CHUNK)], rows[s][c], sems[s]).wait()

  lanes = lax.broadcasted_iota(jnp.int32, (NLANE,), 0)

  def accum_store(i, s):
    zeros = tuple(jnp.zeros((NLANE,), jnp.float32) for _ in range(NVREG))
    ivec = jnp.full((NLANE,), i, jnp.int32)

    def inner(j, acc):
      jvec = jnp.full((NLANE,), j, jnp.int32)
      for c in range(NCHUNK):
        rot = plsc.load_gather(
            idx_v, [ivec, jnp.full((NLANE,), c * CHUNK, jnp.int32) + jvec]) & 15
        perm = (lanes + rot) & 15
        acc = tuple(
            acc[k] + plsc.load_gather(rows[s][c], [jvec, k * NLANE + perm])
            for k in range(NVREG))
      return acc
    acc = lax.fori_loop(0, CHUNK, inner, zeros)
    scale = jnp.float32(1.0 / HIST)
    for k in range(NVREG):
      out_v[i, pl.ds(k * NLANE, NLANE)] = acc[k] * scale

  start_row(0, 0)

  def body(p, _):
    i0 = 2 * p
    start_row(i0 + 1, 1)
    wait_row(0)
    accum_store(i0, 0)

    @pl.when(i0 + 2 < BPW)
    def _():
      start_row(i0 + 2, 0)

    wait_row(1)
    accum_store(i0 + 1, 1)
    return 0

  lax.fori_loop(0, BPW // 2, body, 0)
  pltpu.sync_copy(out_v, out_hbm.at[pl.ds(base, BPW)])


@functools.partial(jax.jit, static_argnums=())
def _sc_gather_mean(idx2d, table):
  mesh = plsc.VectorSubcoreMesh(core_axis_name="c", subcore_axis_name="s")
  return pl.kernel(
      _sc_body,
      out_type=jax.ShapeDtypeStruct((B, D), jnp.float32),
      mesh=mesh,
      compiler_params=pltpu.CompilerParams(use_tc_tiling_on_sc=False,
                                           needs_layout_passes=False),
      scratch_types=(
          [pltpu.VMEM((BPW, HISTP), jnp.int32)]
          + [pltpu.VMEM((CHUNK, D), jnp.float32) for _ in range(2 * NCHUNK)]
          + [pltpu.VMEM((BPW, D), jnp.float32)]
          + [pltpu.SemaphoreType.DMA] * 2
      ),
  )(idx2d, table)


def _mlp_body(x_ref, w1_ref, b1_ref, w2_ref, b2_ref, o_ref):
  x = x_ref[...]
  h = jnp.dot(x, w1_ref[...], preferred_element_type=jnp.float32)
  h = jnp.maximum(h + b1_ref[...], 0.0)
  o_ref[...] = jnp.dot(h, w2_ref[...],
                       preferred_element_type=jnp.float32) + b2_ref[...]


def _mlp(avg, W1, b1, W2, b2):
  return pl.pallas_call(
      _mlp_body,
      out_shape=jax.ShapeDtypeStruct((B, b2.shape[-1]), jnp.float32),
  )(avg, W1, b1, W2, b2)


def kernel(word_indices, table, W1, b1, W2, b2):
  idx_pad = jnp.pad(word_indices.astype(jnp.int32), ((0, 0), (0, HISTP - HIST)))
  table_flat = _sc_transpose(table.T)
  avg = _sc_gather_mean(idx_pad, table_flat.reshape(V, D))
  return _mlp(avg, W1, b1.reshape(1, -1), W2, b2.reshape(1, -1))


# speedup vs baseline: 1.3231x; 1.3231x over previous
import functools

import jax
import jax.numpy as jnp
from jax import lax
from jax.experimental import pallas as pl
from jax.experimental.pallas import tpu as pltpu
from jax.experimental.pallas import tpu_sc as plsc

B = 4096
HIST = 200
D = 64
V = 1000000
NC = 2
NS = 16
NW = NC * NS
BPW = B // NW
CHUNK = 40
NCHUNK = HIST // CHUNK
NLANE = 16
NVREG = D // NLANE
HISTP = 256

BLK = 128
FULLBLK = V // BLK
TAIL = V - FULLBLK * BLK
BLK_PER_W = -(-FULLBLK // NW)


def _tr_body(tt_hbm, out_hbm, in_a, in_b, tail_in, out_a, out_b, tail_out,
             sem_ia, sem_ib, sem_oa, sem_ob):
  cid = lax.axis_index("c")
  sid = lax.axis_index("s")
  wid = sid * NC + cid
  base = wid * BLK_PER_W
  nblk = jnp.minimum(BLK_PER_W, jnp.maximum(0, FULLBLK - base))

  ins = (in_a, in_b)
  outs = (out_a, out_b)
  sem_i = (sem_ia, sem_ib)
  sem_o = (sem_oa, sem_ob)
  lanes = lax.broadcasted_iota(jnp.int32, (NLANE,), 0)

  def start_in(t, s):
    @pl.when(t < nblk)
    def _():
      pltpu.async_copy(tt_hbm.at[:, pl.ds((base + t) * BLK, BLK)],
                       ins[s], sem_i[s])

  def wait_in(s):
    pltpu.make_async_copy(tt_hbm.at[:, pl.ds(0, BLK)], ins[s], sem_i[s]).wait()

  def wait_out(s):
    pltpu.make_async_copy(outs[s], out_hbm.at[pl.ds(0, BLK * D)],
                          sem_o[s]).wait()

  iota64 = D * lanes

  def transpose(s):
    @functools.partial(plsc.parallel_loop, 0, D, unroll=4)
    def _(d):
      vbase = iota64 + ((d + lanes) & 15) + (d - (d & 15))
      for g in range(BLK // NLANE):
        vec = ins[s][d, pl.ds(g * NLANE, NLANE)]
        plsc.store_scatter(outs[s], [vbase + g * (NLANE * D)], vec)

  start_in(0, 0)

  def body(p, _):
    for sbuf in range(2):
      t = 2 * p + sbuf

      @pl.when(t < nblk)
      def _():
        start_in(t + 1, 1 - sbuf)
        wait_in(sbuf)

        @pl.when(t >= 2)
        def _():
          wait_out(sbuf)

        transpose(sbuf)
        pltpu.async_copy(outs[sbuf],
                         out_hbm.at[pl.ds((base + t) * BLK * D, BLK * D)],
                         sem_o[sbuf])
    return 0

  lax.fori_loop(0, (BLK_PER_W + 1) // 2, body, 0)
  wait_out(0)
  wait_out(1)

  @pl.when(wid == NW - 1)
  def _():
    pltpu.sync_copy(tt_hbm.at[:, pl.ds(FULLBLK * BLK, TAIL)], tail_in)

    def drow(d, _):
      vbase = iota64 + ((d + lanes) & 15) + (d - (d & 15))
      for g in range(TAIL // NLANE):
        vec = tail_in[d, pl.ds(g * NLANE, NLANE)]
        plsc.store_scatter(tail_out, [vbase + g * (NLANE * D)], vec)
      return 0
    lax.fori_loop(0, D, drow, 0)
    pltpu.sync_copy(tail_out, out_hbm.at[pl.ds(FULLBLK * BLK * D, TAIL * D)])


@functools.partial(jax.jit, static_argnums=())
def _sc_transpose(table_t):
  mesh = plsc.VectorSubcoreMesh(core_axis_name="c", subcore_axis_name="s")
  return pl.kernel(
      _tr_body,
      out_type=jax.ShapeDtypeStruct((V * D,), jnp.float32),
      mesh=mesh,
      compiler_params=pltpu.CompilerParams(use_tc_tiling_on_sc=True,
                                           needs_layout_passes=False),
      scratch_types=(
          [pltpu.VMEM((D, BLK), jnp.float32) for _ in range(2)]
          + [pltpu.VMEM((D, TAIL), jnp.float32)]
          + [pltpu.VMEM((BLK * D,), jnp.float32) for _ in range(2)]
          + [pltpu.VMEM((TAIL * D,), jnp.float32)]
          + [pltpu.SemaphoreType.DMA] * 4
      ),
  )(table_t)


def _sc_body(idx_hbm, table_hbm, out_hbm, idx_v, *rest):
  rows_flat = rest[:2 * NCHUNK]
  out_v = rest[2 * NCHUNK]
  sems = rest[2 * NCHUNK + 1:]
  rows = (rows_flat[:NCHUNK], rows_flat[NCHUNK:])

  cid = lax.axis_index("c")
  sid = lax.axis_index("s")
  wid = sid * NC + cid
  base = wid * BPW

  pltpu.sync_copy(idx_hbm.at[pl.ds(base, BPW)], idx_v)

  def start_row(i, s):
    for c in range(NCHUNK):
      pltpu.async_copy(
          table_hbm.at[idx_v.at[i, pl.ds(c * CHUNK, CHUNK)]],
          rows[s][c], sems[s])

  def wait_row(s):
    for c in range(NCHUNK):
      pltpu.make_async_copy(
          table_hbm.at[pl.ds(0, CHUNK)], rows[s][c], sems[s]).wait()

  lanes = lax.broadcasted_iota(jnp.int32, (NLANE,), 0)

  def accum_store(i, s):
    zeros = tuple(jnp.zeros((NLANE,), jnp.float32) for _ in range(NVREG))
    ivec = jnp.full((NLANE,), i, jnp.int32)

    def inner(j, acc):
      jvec = jnp.full((NLANE,), j, jnp.int32)
      for c in range(NCHUNK):
        rot = plsc.load_gather(idx_v, [ivec,
                                       jnp.full((NLANE,), c * CHUNK + j,
                                                jnp.int32)]) & 15
        perm = (lanes + rot) & 15
        acc = tuple(
            acc[k] + plsc.load_gather(rows[s][c], [jvec, k * NLANE + perm])
            for k in range(NVREG))
      return acc
    acc = lax.fori_loop(0, CHUNK, inner, zeros)
    scale = jnp.float32(1.0 / HIST)
    for k in range(NVREG):
      out_v[i, pl.ds(k * NLANE, NLANE)] = acc[k] * scale

  start_row(0, 0)

  def body(p, _):
    i0 = 2 * p
    start_row(i0 + 1, 1)
    wait_row(0)
    accum_store(i0, 0)

    @pl.when(i0 + 2 < BPW)
    def _():
      start_row(i0 + 2, 0)

    wait_row(1)
    accum_store(i0 + 1, 1)
    return 0

  lax.fori_loop(0, BPW // 2, body, 0)
  pltpu.sync_copy(out_v, out_hbm.at[pl.ds(base, BPW)])


@functools.partial(jax.jit, static_argnums=())
def _sc_gather_mean(idx2d, table):
  mesh = plsc.VectorSubcoreMesh(core_axis_name="c", subcore_axis_name="s")
  return pl.kernel(
      _sc_body,
      out_type=jax.ShapeDtypeStruct((B, D), jnp.float32),
      mesh=mesh,
      compiler_params=pltpu.CompilerParams(use_tc_tiling_on_sc=False,
                                           needs_layout_passes=False),
      scratch_types=(
          [pltpu.VMEM((BPW, HISTP), jnp.int32)]
          + [pltpu.VMEM((CHUNK, D), jnp.float32) for _ in range(2 * NCHUNK)]
          + [pltpu.VMEM((BPW, D), jnp.float32)]
          + [pltpu.SemaphoreType.DMA] * 2
      ),
  )(idx2d, table)


def _mlp_body(x_ref, w1_ref, b1_ref, w2_ref, b2_ref, o_ref):
  x = x_ref[...]
  h = jnp.dot(x, w1_ref[...], preferred_element_type=jnp.float32)
  h = jnp.maximum(h + b1_ref[...], 0.0)
  o_ref[...] = jnp.dot(h, w2_ref[...],
                       preferred_element_type=jnp.float32) + b2_ref[...]


def _mlp(avg, W1, b1, W2, b2):
  return pl.pallas_call(
      _mlp_body,
      out_shape=jax.ShapeDtypeStruct((B, b2.shape[-1]), jnp.float32),
  )(avg, W1, b1, W2, b2)


def kernel(word_indices, table, W1, b1, W2, b2):
  idx_pad = jnp.pad(word_indices.astype(jnp.int32), ((0, 0), (0, HISTP - HIST)))
  table_flat = _sc_transpose(table.T)
  avg = _sc_gather_mean(idx_pad, table_flat.reshape(V, D))
  return _mlp(avg, W1, b1.reshape(1, -1), W2, b2.reshape(1, -1))
